# baseline (speedup 1.0000x reference)
import functools

import jax
import jax.numpy as jnp
from jax import lax
from jax.experimental import pallas as pl
from jax.experimental.pallas import tpu as pltpu
from jax.experimental.pallas import tpu_sc as plsc

D = 128
S = 10
C = 32
NW = 32
LANES = 16


def _build_sc_kernel(n_pad: int):
    chunks_per_w = n_pad // (NW * C)
    mesh = plsc.VectorSubcoreMesh(core_axis_name="c", subcore_axis_name="s")

    @functools.partial(
        pl.kernel,
        mesh=mesh,
        out_type=jax.ShapeDtypeStruct((n_pad, D), jnp.float32),
        scratch_types=[
            pltpu.VMEM((S, C), jnp.int32),
            pltpu.VMEM((S * C, D), jnp.float32),
            pltpu.VMEM((C, D), jnp.float32),
            pltpu.SemaphoreType.DMA,
        ],
    )
    def sc_kernel(feat_hbm, idx_hbm, out_hbm, idx_v, rows_v, out_v, sem):
        cid = lax.axis_index("c")
        sid = lax.axis_index("s")
        n_chunks = n_pad // C
        k0 = (n_chunks * 41) // (16 * (41 + 57))
        k1 = n_chunks // 16 - k0
        my_k = jnp.where(cid == 0, k0, k1)
        base = jnp.where(cid == 0, sid * k0, 16 * k0 + sid * k1)

        def chunk_body(j, carry):
            chunk = base + j
            pltpu.sync_copy(idx_hbm.at[chunk], idx_v)
            handles = []
            for s in range(S):
                handles.append(
                    pltpu.async_copy(
                        feat_hbm.at[idx_v.at[s]], rows_v.at[pl.ds(s * C, C)], sem
                    )
                )
            for h in handles:
                h.wait()

            def node_body(i, c2):
                for g in range(D // LANES):
                    col = pl.ds(g * LANES, LANES)
                    acc = rows_v[i, col]
                    for s in range(1, S):
                        acc = jnp.maximum(acc, rows_v[s * C + i, col])
                    out_v[i, col] = acc
                return c2

            lax.fori_loop(0, C, node_body, 0)
            pltpu.sync_copy(out_v, out_hbm.at[pl.ds(chunk * C, C)])
            return carry

        lax.fori_loop(0, my_k, chunk_body, 0)

    return sc_kernel


def kernel(features, nodes, nbrs, num_sample):
    del nodes, num_sample
    n = features.shape[0]
    n_pad = ((n + NW * C - 1) // (NW * C)) * (NW * C)
    nbrs32 = jnp.pad(nbrs.astype(jnp.int32), ((0, n_pad - n), (0, 0)))
    idx_chunks = nbrs32.reshape(n_pad // C, C, S).transpose(0, 2, 1)
    out = _build_sc_kernel(n_pad)(features, idx_chunks)
    return out[:n]


# speedup vs baseline: 1.1911x; 1.1911x over previous
import functools

import jax
import jax.numpy as jnp
from jax import lax
from jax.experimental import pallas as pl
from jax.experimental.pallas import tpu as pltpu
from jax.experimental.pallas import tpu_sc as plsc

D = 128
S = 10
C = 32
NW = 32
LANES = 16


def _build_sc_kernel(n_pad: int):
    chunks_per_w = n_pad // (NW * C)
    mesh = plsc.VectorSubcoreMesh(core_axis_name="c", subcore_axis_name="s")

    @functools.partial(
        pl.kernel,
        mesh=mesh,
        out_type=jax.ShapeDtypeStruct((n_pad, D), jnp.float32),
        scratch_types=[
            pltpu.VMEM((S, C), jnp.int32),
            pltpu.VMEM((S * C, D), jnp.float32),
            pltpu.VMEM((C, D), jnp.float32),
            pltpu.SemaphoreType.DMA,
        ],
    )
    def sc_kernel(feat_hbm, idx_hbm, out_hbm, idx_v, rows_v, out_v, sem):
        cid = lax.axis_index("c")
        sid = lax.axis_index("s")
        n_chunks = n_pad // C
        k0 = (n_chunks * 57) // (16 * (41 + 57))
        k1 = n_chunks // 16 - k0
        my_k = jnp.where(cid == 0, k0, k1)
        base = jnp.where(cid == 0, sid * k0, 16 * k0 + sid * k1)

        def chunk_body(j, carry):
            chunk = base + j
            pltpu.sync_copy(idx_hbm.at[chunk], idx_v)
            handles = []
            for s in range(S):
                handles.append(
                    pltpu.async_copy(
                        feat_hbm.at[idx_v.at[s]], rows_v.at[pl.ds(s * C, C)], sem
                    )
                )
            for h in handles:
                h.wait()

            def node_body(i, c2):
                for g in range(D // LANES):
                    col = pl.ds(g * LANES, LANES)
                    acc = rows_v[i, col]
                    for s in range(1, S):
                        acc = jnp.maximum(acc, rows_v[s * C + i, col])
                    out_v[i, col] = acc
                return c2

            lax.fori_loop(0, C, node_body, 0)
            pltpu.sync_copy(out_v, out_hbm.at[pl.ds(chunk * C, C)])
            return carry

        lax.fori_loop(0, my_k, chunk_body, 0)

    return sc_kernel


def kernel(features, nodes, nbrs, num_sample):
    del nodes, num_sample
    n = features.shape[0]
    n_pad = ((n + NW * C - 1) // (NW * C)) * (NW * C)
    nbrs32 = jnp.pad(nbrs.astype(jnp.int32), ((0, n_pad - n), (0, 0)))
    idx_chunks = nbrs32.reshape(n_pad // C, C, S).transpose(0, 2, 1)
    out = _build_sc_kernel(n_pad)(features, idx_chunks)
    return out[:n]
